# baseline (speedup 1.0000x reference)
import functools

import jax
import jax.numpy as jnp
from jax import lax
from jax.experimental import pallas as pl
from jax.experimental.pallas import tpu as pltpu
from jax.experimental.pallas import tpu_sc as plsc

_NC = 2
_NS = 16
_NW = _NC * _NS
_CH = 80
_G = 64


def _pad_up(n, m):
    return (n + m - 1) // m * m


_KB = 4


def _deg_call(npad, kc):
    mesh = plsc.VectorSubcoreMesh(core_axis_name="c", subcore_axis_name="s")
    rpt = npad // _NS
    nblk = kc // _KB

    half = nblk // 2

    @functools.partial(
        pl.kernel,
        out_type=jax.ShapeDtypeStruct((_NC, npad), jnp.float32),
        mesh=mesh,
        scratch_types=[
            pltpu.VMEM((2, _KB, _CH), jnp.int32),
            pltpu.VMEM((_CH,), jnp.float32),
            pltpu.VMEM_SHARED((npad,), jnp.float32),
            pltpu.SemaphoreType.DMA,
            pltpu.SemaphoreType.DMA,
            pltpu.SemaphoreType.DMA,
            pltpu.SemaphoreType.DMA,
        ],
    )
    def deg(dst_hbm, z1_hbm, out_hbm, dst_v, ones_v, acc_sh,
            ssemA, ssemB, isemA, isemB):
        cid = lax.axis_index("c")
        sid = lax.axis_index("s")
        wid = sid * _NC + cid
        for j in range(_CH // 16):
            ones_v[pl.ds(j * 16, 16)] = jnp.ones((16,), jnp.float32)
        pltpu.sync_copy(z1_hbm.at[pl.ds(sid * rpt, rpt)],
                        acc_sh.at[pl.ds(sid * rpt, rpt)])
        plsc.subcore_barrier()
        ssems = (ssemA, ssemB)
        isems = (isemA, isemB)

        def load_idx(blk, ring):
            pltpu.async_copy(dst_hbm.at[wid, pl.ds(blk * _KB, _KB)],
                             dst_v.at[ring], isems[ring])

        def drain_idx(ring):
            pltpu.make_async_copy(dst_hbm.at[wid, pl.ds(0, _KB)],
                                  dst_v.at[ring], isems[ring]).wait()

        def drain_scat(ring):
            for _ in range(_KB):
                pltpu.make_async_copy(
                    ones_v, acc_sh.at[dst_v.at[ring, 0]], ssems[ring]).wait()

        load_idx(0, 0)
        load_idx(1, 1)

        def body(t, carry):
            for ring in range(2):
                drain_idx(ring)
                for j in range(_KB):
                    pltpu.async_copy(ones_v, acc_sh.at[dst_v.at[ring, j]],
                                     ssems[ring], add=True)

            @pl.when(t + 1 < half)
            def _():
                for ring in range(2):
                    drain_scat(ring)
                    load_idx(2 * t + 2 + ring, ring)
            return carry

        lax.fori_loop(0, half, body, 0)
        drain_scat(0)
        drain_scat(1)
        plsc.subcore_barrier()
        pltpu.sync_copy(acc_sh.at[pl.ds(sid * rpt, rpt)],
                        out_hbm.at[cid, pl.ds(sid * rpt, rpt)])

    return deg


def _spmm_call(npad, kc, d):
    mesh = plsc.VectorSubcoreMesh(core_axis_name="c", subcore_axis_name="s")
    rpt = npad // _NS
    nblk = kc // _KB
    half = nblk // 2

    @functools.partial(
        pl.kernel,
        out_type=jax.ShapeDtypeStruct((_NC, npad, d), jnp.float32),
        mesh=mesh,
        scratch_types=[
            pltpu.VMEM((2, _KB, _CH), jnp.int32),
            pltpu.VMEM((2, _KB, _CH), jnp.int32),
            pltpu.VMEM((2, _CH, d), jnp.float32),
            pltpu.VMEM_SHARED((npad, d), jnp.float32),
            pltpu.SemaphoreType.DMA,
            pltpu.SemaphoreType.DMA,
            pltpu.SemaphoreType.DMA,
            pltpu.SemaphoreType.DMA,
            pltpu.SemaphoreType.DMA,
            pltpu.SemaphoreType.DMA,
        ],
    )
    def spmm(src_hbm, dst_hbm, hs_hbm, zr_hbm, out_hbm,
             src_v, dst_v, rows_v, acc_sh,
             gsem0, gsem1, ssem0, ssem1, isemA, isemB):
        cid = lax.axis_index("c")
        sid = lax.axis_index("s")
        wid = sid * _NC + cid
        pltpu.sync_copy(zr_hbm, acc_sh.at[pl.ds(sid * rpt, rpt)])
        plsc.subcore_barrier()
        gsems = (gsem0, gsem1)
        ssems = (ssem0, ssem1)
        isems = (isemA, isemB)

        def load_idx(blk, ring):
            pltpu.async_copy(src_hbm.at[wid, pl.ds(blk * _KB, _KB)],
                             src_v.at[ring], isems[ring])
            pltpu.async_copy(dst_hbm.at[wid, pl.ds(blk * _KB, _KB)],
                             dst_v.at[ring], isems[ring])

        def drain_idx(ring):
            pltpu.make_async_copy(src_hbm.at[wid, pl.ds(0, _KB)],
                                  src_v.at[ring], isems[ring]).wait()
            pltpu.make_async_copy(dst_hbm.at[wid, pl.ds(0, _KB)],
                                  dst_v.at[ring], isems[ring]).wait()

        def fire_gather(ring, j, buf):
            pltpu.async_copy(hs_hbm.at[src_v.at[ring, j]], rows_v.at[buf],
                             gsems[buf])

        def drain_gather(buf):
            pltpu.make_async_copy(hs_hbm.at[src_v.at[0, 0]], rows_v.at[buf],
                                  gsems[buf]).wait()

        def fire_scat(ring, j, buf):
            pltpu.async_copy(rows_v.at[buf], acc_sh.at[dst_v.at[ring, j]],
                             ssems[buf], add=True)

        def drain_scat(buf):
            pltpu.make_async_copy(rows_v.at[buf],
                                  acc_sh.at[dst_v.at[0, 0]],
                                  ssems[buf]).wait()

        load_idx(0, 0)
        drain_idx(0)
        load_idx(1, 1)
        fire_gather(0, 0, 0)

        def body(t, carry):
            for s in range(2 * _KB):
                ring = s // _KB
                j = s % _KB
                buf = s % 2
                nring = (s + 1) // _KB
                nj = (s + 1) % _KB
                if s + 1 < 2 * _KB:
                    if nj == 0:
                        drain_idx(nring)
                    fire_gather(nring, nj, (s + 1) % 2)
                else:
                    @pl.when(t + 1 < half)
                    def _():
                        drain_idx(0)
                        fire_gather(0, 0, (s + 1) % 2)
                drain_gather(buf)
                pltpu.sync_copy(rows_v.at[buf], acc_sh.at[dst_v.at[ring, j]],
                                add=True)
                if s == _KB - 1:
                    @pl.when(t + 1 < half)
                    def _():
                        load_idx(2 * t + 2, 0)
                if s == 2 * _KB - 1:
                    @pl.when(t + 1 < half)
                    def _():
                        load_idx(2 * t + 3, 1)
            return carry

        lax.fori_loop(0, half, body, 0)
        plsc.subcore_barrier()
        pltpu.sync_copy(acc_sh.at[pl.ds(sid * rpt, rpt)],
                        out_hbm.at[cid, pl.ds(sid * rpt, rpt)])

    return spmm


def _disb(degc_ref, bm, h):
    return jnp.broadcast_to(lax.rsqrt(degc_ref[...]), (bm, h))


def _mm_scale_call(n, npad, d, h, bm):
    def body(x_ref, w_ref, deg_ref, hs_ref):
        disb = _disb(deg_ref, bm, h)
        mm = jnp.dot(x_ref[...], w_ref[...], preferred_element_type=jnp.float32)
        hs_ref[...] = mm * disb

    return pl.pallas_call(
        body,
        grid=(n // bm,),
        in_specs=[
            pl.BlockSpec((bm, d), lambda i: (i, 0)),
            pl.BlockSpec((d, h), lambda i: (0, 0)),
            pl.BlockSpec((bm, 1), lambda i: (i, 0)),
        ],
        out_specs=pl.BlockSpec((bm, h), lambda i: (i, 0)),
        out_shape=jax.ShapeDtypeStruct((npad, h), jnp.float32),
    )


def _post(acc_ref, hs_ref, disb, b_ref, g_ref, be_ref):
    z = (acc_ref[0] + acc_ref[1] + hs_ref[...]) * disb + b_ref[...]
    mu = jnp.mean(z, axis=1, keepdims=True)
    zc = z - mu
    var = jnp.mean(zc * zc, axis=1, keepdims=True)
    zn = zc * lax.rsqrt(var + 1e-5) * g_ref[...] + be_ref[...]
    return jnp.maximum(zn, 0.0)


def _post_mm_call(n, npad, h, bm):
    def body(acc_ref, hs_ref, deg_ref, b_ref, g_ref, be_ref, w_ref, o_ref):
        disb = _disb(deg_ref, bm, h)
        z = _post(acc_ref, hs_ref, disb, b_ref, g_ref, be_ref)
        o_ref[...] = jnp.dot(z, w_ref[...],
                             preferred_element_type=jnp.float32) * disb

    return pl.pallas_call(
        body,
        grid=(n // bm,),
        in_specs=[
            pl.BlockSpec((_NC, bm, h), lambda i: (0, i, 0)),
            pl.BlockSpec((bm, h), lambda i: (i, 0)),
            pl.BlockSpec((bm, 1), lambda i: (i, 0)),
            pl.BlockSpec((1, h), lambda i: (0, 0)),
            pl.BlockSpec((1, h), lambda i: (0, 0)),
            pl.BlockSpec((1, h), lambda i: (0, 0)),
            pl.BlockSpec((h, h), lambda i: (0, 0)),
        ],
        out_specs=pl.BlockSpec((bm, h), lambda i: (i, 0)),
        out_shape=jax.ShapeDtypeStruct((npad, h), jnp.float32),
    )


def _post_pool_call(n, npad, h, c, bm):
    nb = n // bm

    def body(acc_ref, hs_ref, deg_ref, b_ref, g_ref, be_ref, bt_ref,
             wl_ref, bl_ref, o_ref, sums_ref, cnts_ref):
        i = pl.program_id(0)
        z = _post(acc_ref, hs_ref, _disb(deg_ref, bm, h), b_ref, g_ref, be_ref)
        seg = lax.broadcasted_iota(jnp.int32, (bm, _G), 1)
        mask = (bt_ref[...] == seg).astype(jnp.float32)

        @pl.when(i == 0)
        def _():
            sums_ref[...] = jnp.zeros((_G, h), jnp.float32)
            cnts_ref[...] = jnp.zeros((_G, h), jnp.float32)

        dn = (((0,), (0,)), ((), ()))
        sums_ref[...] += lax.dot_general(
            mask, z, dn, preferred_element_type=jnp.float32)
        cnts_ref[...] += jnp.broadcast_to(
            lax.dot_general(mask, jnp.ones((bm, 1), jnp.float32), dn,
                            preferred_element_type=jnp.float32), (_G, h))

        @pl.when(i == nb - 1)
        def _():
            pooled = sums_ref[...] / jnp.maximum(cnts_ref[...], 1.0)
            o_ref[...] = jnp.dot(pooled, wl_ref[...],
                                 preferred_element_type=jnp.float32) + bl_ref[...]

    return pl.pallas_call(
        body,
        grid=(nb,),
        in_specs=[
            pl.BlockSpec((_NC, bm, h), lambda i: (0, i, 0)),
            pl.BlockSpec((bm, h), lambda i: (i, 0)),
            pl.BlockSpec((bm, 1), lambda i: (i, 0)),
            pl.BlockSpec((1, h), lambda i: (0, 0)),
            pl.BlockSpec((1, h), lambda i: (0, 0)),
            pl.BlockSpec((1, h), lambda i: (0, 0)),
            pl.BlockSpec((bm, 1), lambda i: (i, 0)),
            pl.BlockSpec((h, c), lambda i: (0, 0)),
            pl.BlockSpec((1, c), lambda i: (0, 0)),
        ],
        out_specs=pl.BlockSpec((_G, c), lambda i: (0, 0)),
        out_shape=jax.ShapeDtypeStruct((_G, c), jnp.float32),
        scratch_shapes=[
            pltpu.VMEM((_G, h), jnp.float32),
            pltpu.VMEM((_G, h), jnp.float32),
        ],
    )


def kernel(x, edge, batch, W1, b1, W2, b2, gamma, beta, Wl, bl):
    n, d = x.shape
    e = edge.shape[1]
    h = W1.shape[1]
    c = Wl.shape[1]

    npad = _pad_up(n + 1, _NS * 128)
    ech = _NW * _CH
    kc = _pad_up(e, 2 * _KB * ech) // ech
    epad = kc * ech
    nscr = npad - n

    npd = jnp.arange(epad - e, dtype=jnp.int32)
    src_ch = jnp.concatenate([edge[0], npd % n]).reshape(_NW, kc, _CH)
    dst_ch = jnp.concatenate([edge[1], n + npd % nscr]).reshape(_NW, kc, _CH)

    batchc = batch.reshape(n, 1)
    rpt = npad // _NS
    z1 = jnp.zeros((npad,), jnp.float32)
    zr = jnp.zeros((rpt, h), jnp.float32)

    b1r = b1.reshape(1, h)
    b2r = b2.reshape(1, h)
    gr = gamma.reshape(1, h)
    ber = beta.reshape(1, h)
    blr = bl.reshape(1, c)

    bm = 2000

    deg = _deg_call(npad, kc)(dst_ch, z1)
    degc = (deg[0, :n] + deg[1, :n] + 1.0).reshape(n, 1)
    hs1 = _mm_scale_call(n, npad, d, h, bm)(x, W1, degc)
    acc1 = _spmm_call(npad, kc, h)(src_ch, dst_ch, hs1, zr)
    hs2 = _post_mm_call(n, npad, h, bm)(acc1, hs1, degc, b1r, gr, ber, W2)
    acc2 = _spmm_call(npad, kc, h)(src_ch, dst_ch, hs2, zr)
    out = _post_pool_call(n, npad, h, c, bm)(
        acc2, hs2, degc, b2r, gr, ber, batchc, Wl, blr)
    return out


# speedup vs baseline: 1.0093x; 1.0093x over previous
import functools

import jax
import jax.numpy as jnp
from jax import lax
from jax.experimental import pallas as pl
from jax.experimental.pallas import tpu as pltpu
from jax.experimental.pallas import tpu_sc as plsc

_NC = 2
_NS = 16
_NW = _NC * _NS
_CH = 80
_G = 64


def _pad_up(n, m):
    return (n + m - 1) // m * m


_KB = 4


def _deg_call(npad, kc):
    mesh = plsc.VectorSubcoreMesh(core_axis_name="c", subcore_axis_name="s")
    rpt = npad // _NS
    nblk = kc // _KB

    half = nblk // 2

    @functools.partial(
        pl.kernel,
        out_type=jax.ShapeDtypeStruct((_NC, npad), jnp.float32),
        mesh=mesh,
        scratch_types=[
            pltpu.VMEM((2, _KB, _CH), jnp.int32),
            pltpu.VMEM((_CH,), jnp.float32),
            pltpu.VMEM_SHARED((npad,), jnp.float32),
            pltpu.SemaphoreType.DMA,
            pltpu.SemaphoreType.DMA,
            pltpu.SemaphoreType.DMA,
            pltpu.SemaphoreType.DMA,
        ],
    )
    def deg(dst_hbm, z1_hbm, out_hbm, dst_v, ones_v, acc_sh,
            ssemA, ssemB, isemA, isemB):
        cid = lax.axis_index("c")
        sid = lax.axis_index("s")
        wid = sid * _NC + cid
        for j in range(_CH // 16):
            ones_v[pl.ds(j * 16, 16)] = jnp.ones((16,), jnp.float32)
        pltpu.sync_copy(z1_hbm.at[pl.ds(sid * rpt, rpt)],
                        acc_sh.at[pl.ds(sid * rpt, rpt)])
        plsc.subcore_barrier()
        ssems = (ssemA, ssemB)
        isems = (isemA, isemB)

        def load_idx(blk, ring):
            pltpu.async_copy(dst_hbm.at[wid, pl.ds(blk * _KB, _KB)],
                             dst_v.at[ring], isems[ring])

        def drain_idx(ring):
            pltpu.make_async_copy(dst_hbm.at[wid, pl.ds(0, _KB)],
                                  dst_v.at[ring], isems[ring]).wait()

        def drain_scat(ring):
            for _ in range(_KB):
                pltpu.make_async_copy(
                    ones_v, acc_sh.at[dst_v.at[ring, 0]], ssems[ring]).wait()

        load_idx(0, 0)
        load_idx(1, 1)

        def body(t, carry):
            for ring in range(2):
                drain_idx(ring)
                for j in range(_KB):
                    pltpu.async_copy(ones_v, acc_sh.at[dst_v.at[ring, j]],
                                     ssems[ring], add=True)

            @pl.when(t + 1 < half)
            def _():
                for ring in range(2):
                    drain_scat(ring)
                    load_idx(2 * t + 2 + ring, ring)
            return carry

        lax.fori_loop(0, half, body, 0)
        drain_scat(0)
        drain_scat(1)
        plsc.subcore_barrier()
        pltpu.sync_copy(acc_sh.at[pl.ds(sid * rpt, rpt)],
                        out_hbm.at[cid, pl.ds(sid * rpt, rpt)])

    return deg


def _spmm_call(npad, kc, d):
    mesh = plsc.VectorSubcoreMesh(core_axis_name="c", subcore_axis_name="s")
    rpt = npad // _NS
    nblk = kc // _KB
    half = nblk // 2

    @functools.partial(
        pl.kernel,
        out_type=jax.ShapeDtypeStruct((_NC, npad, d), jnp.float32),
        mesh=mesh,
        scratch_types=[
            pltpu.VMEM((2, _KB, _CH), jnp.int32),
            pltpu.VMEM((2, _KB, _CH), jnp.int32),
            pltpu.VMEM((2, _CH, d), jnp.float32),
            pltpu.VMEM_SHARED((npad, d), jnp.float32),
            pltpu.SemaphoreType.DMA,
            pltpu.SemaphoreType.DMA,
            pltpu.SemaphoreType.DMA,
            pltpu.SemaphoreType.DMA,
            pltpu.SemaphoreType.DMA,
            pltpu.SemaphoreType.DMA,
        ],
    )
    def spmm(src_hbm, dst_hbm, hs_hbm, zr_hbm, out_hbm,
             src_v, dst_v, rows_v, acc_sh,
             gsem0, gsem1, ssem0, ssem1, isemA, isemB):
        cid = lax.axis_index("c")
        sid = lax.axis_index("s")
        wid = sid * _NC + cid
        pltpu.sync_copy(zr_hbm.at[sid], acc_sh.at[pl.ds(sid * rpt, rpt)])
        plsc.subcore_barrier()
        gsems = (gsem0, gsem1)
        ssems = (ssem0, ssem1)
        isems = (isemA, isemB)

        def load_idx(blk, ring):
            pltpu.async_copy(src_hbm.at[wid, pl.ds(blk * _KB, _KB)],
                             src_v.at[ring], isems[ring])
            pltpu.async_copy(dst_hbm.at[wid, pl.ds(blk * _KB, _KB)],
                             dst_v.at[ring], isems[ring])

        def drain_idx(ring):
            pltpu.make_async_copy(src_hbm.at[wid, pl.ds(0, _KB)],
                                  src_v.at[ring], isems[ring]).wait()
            pltpu.make_async_copy(dst_hbm.at[wid, pl.ds(0, _KB)],
                                  dst_v.at[ring], isems[ring]).wait()

        def fire_gather(ring, j, buf):
            pltpu.async_copy(hs_hbm.at[src_v.at[ring, j]], rows_v.at[buf],
                             gsems[buf])

        def drain_gather(buf):
            pltpu.make_async_copy(hs_hbm.at[src_v.at[0, 0]], rows_v.at[buf],
                                  gsems[buf]).wait()

        def fire_scat(ring, j, buf):
            pltpu.async_copy(rows_v.at[buf], acc_sh.at[dst_v.at[ring, j]],
                             ssems[buf], add=True)

        def drain_scat(buf):
            pltpu.make_async_copy(rows_v.at[buf],
                                  acc_sh.at[dst_v.at[0, 0]],
                                  ssems[buf]).wait()

        load_idx(0, 0)
        drain_idx(0)
        load_idx(1, 1)
        fire_gather(0, 0, 0)

        def body(t, carry):
            for s in range(2 * _KB):
                ring = s // _KB
                j = s % _KB
                buf = s % 2
                nring = (s + 1) // _KB
                nj = (s + 1) % _KB
                if s + 1 < 2 * _KB:
                    if nj == 0:
                        drain_idx(nring)
                    fire_gather(nring, nj, (s + 1) % 2)
                else:
                    @pl.when(t + 1 < half)
                    def _():
                        drain_idx(0)
                        fire_gather(0, 0, (s + 1) % 2)
                drain_gather(buf)
                pltpu.sync_copy(rows_v.at[buf], acc_sh.at[dst_v.at[ring, j]],
                                add=True)
                if s == _KB - 1:
                    @pl.when(t + 1 < half)
                    def _():
                        load_idx(2 * t + 2, 0)
                if s == 2 * _KB - 1:
                    @pl.when(t + 1 < half)
                    def _():
                        load_idx(2 * t + 3, 1)
            return carry

        lax.fori_loop(0, half, body, 0)
        plsc.subcore_barrier()
        pltpu.sync_copy(acc_sh.at[pl.ds(sid * rpt, rpt)],
                        out_hbm.at[cid, pl.ds(sid * rpt, rpt)])

    return spmm


def _disb(degc_ref, bm, h):
    return jnp.broadcast_to(lax.rsqrt(degc_ref[...]), (bm, h))


def _mm_scale_call(n, npad, d, h, bm):
    def body(x_ref, w_ref, deg_ref, hs_ref):
        disb = _disb(deg_ref, bm, h)
        mm = jnp.dot(x_ref[...], w_ref[...], preferred_element_type=jnp.float32)
        hs_ref[...] = mm * disb

    return pl.pallas_call(
        body,
        grid=(n // bm,),
        in_specs=[
            pl.BlockSpec((bm, d), lambda i: (i, 0)),
            pl.BlockSpec((d, h), lambda i: (0, 0)),
            pl.BlockSpec((bm, 1), lambda i: (i, 0)),
        ],
        out_specs=pl.BlockSpec((bm, h), lambda i: (i, 0)),
        out_shape=jax.ShapeDtypeStruct((npad, h), jnp.float32),
    )


def _post(acc_ref, hs_ref, disb, b_ref, g_ref, be_ref):
    z = (acc_ref[0] + acc_ref[1] + hs_ref[...]) * disb + b_ref[...]
    mu = jnp.mean(z, axis=1, keepdims=True)
    zc = z - mu
    var = jnp.mean(zc * zc, axis=1, keepdims=True)
    zn = zc * lax.rsqrt(var + 1e-5) * g_ref[...] + be_ref[...]
    return jnp.maximum(zn, 0.0)


def _post_mm_call(n, npad, h, bm):
    def body(acc_ref, hs_ref, deg_ref, b_ref, g_ref, be_ref, w_ref, o_ref):
        disb = _disb(deg_ref, bm, h)
        z = _post(acc_ref, hs_ref, disb, b_ref, g_ref, be_ref)
        o_ref[...] = jnp.dot(z, w_ref[...],
                             preferred_element_type=jnp.float32) * disb

    return pl.pallas_call(
        body,
        grid=(n // bm,),
        in_specs=[
            pl.BlockSpec((_NC, bm, h), lambda i: (0, i, 0)),
            pl.BlockSpec((bm, h), lambda i: (i, 0)),
            pl.BlockSpec((bm, 1), lambda i: (i, 0)),
            pl.BlockSpec((1, h), lambda i: (0, 0)),
            pl.BlockSpec((1, h), lambda i: (0, 0)),
            pl.BlockSpec((1, h), lambda i: (0, 0)),
            pl.BlockSpec((h, h), lambda i: (0, 0)),
        ],
        out_specs=pl.BlockSpec((bm, h), lambda i: (i, 0)),
        out_shape=jax.ShapeDtypeStruct((npad, h), jnp.float32),
    )


def _post_pool_call(n, npad, h, c, bm):
    nb = n // bm

    def body(acc_ref, hs_ref, deg_ref, b_ref, g_ref, be_ref, bt_ref,
             wl_ref, bl_ref, o_ref, sums_ref, cnts_ref):
        i = pl.program_id(0)
        z = _post(acc_ref, hs_ref, _disb(deg_ref, bm, h), b_ref, g_ref, be_ref)
        seg = lax.broadcasted_iota(jnp.int32, (bm, _G), 1)
        mask = (bt_ref[...] == seg).astype(jnp.float32)

        @pl.when(i == 0)
        def _():
            sums_ref[...] = jnp.zeros((_G, h), jnp.float32)
            cnts_ref[...] = jnp.zeros((_G, h), jnp.float32)

        dn = (((0,), (0,)), ((), ()))
        sums_ref[...] += lax.dot_general(
            mask, z, dn, preferred_element_type=jnp.float32)
        cnts_ref[...] += jnp.broadcast_to(
            lax.dot_general(mask, jnp.ones((bm, 1), jnp.float32), dn,
                            preferred_element_type=jnp.float32), (_G, h))

        @pl.when(i == nb - 1)
        def _():
            pooled = sums_ref[...] / jnp.maximum(cnts_ref[...], 1.0)
            o_ref[...] = jnp.dot(pooled, wl_ref[...],
                                 preferred_element_type=jnp.float32) + bl_ref[...]

    return pl.pallas_call(
        body,
        grid=(nb,),
        in_specs=[
            pl.BlockSpec((_NC, bm, h), lambda i: (0, i, 0)),
            pl.BlockSpec((bm, h), lambda i: (i, 0)),
            pl.BlockSpec((bm, 1), lambda i: (i, 0)),
            pl.BlockSpec((1, h), lambda i: (0, 0)),
            pl.BlockSpec((1, h), lambda i: (0, 0)),
            pl.BlockSpec((1, h), lambda i: (0, 0)),
            pl.BlockSpec((bm, 1), lambda i: (i, 0)),
            pl.BlockSpec((h, c), lambda i: (0, 0)),
            pl.BlockSpec((1, c), lambda i: (0, 0)),
        ],
        out_specs=pl.BlockSpec((_G, c), lambda i: (0, 0)),
        out_shape=jax.ShapeDtypeStruct((_G, c), jnp.float32),
        scratch_shapes=[
            pltpu.VMEM((_G, h), jnp.float32),
            pltpu.VMEM((_G, h), jnp.float32),
        ],
    )


def kernel(x, edge, batch, W1, b1, W2, b2, gamma, beta, Wl, bl):
    n, d = x.shape
    e = edge.shape[1]
    h = W1.shape[1]
    c = Wl.shape[1]

    npad = _pad_up(n + 1, _NS * 128)
    ech = _NW * _CH
    kc = _pad_up(e, 2 * _KB * ech) // ech
    epad = kc * ech
    nscr = npad - n

    npd = jnp.arange(epad - e, dtype=jnp.int32)
    src_ch = jnp.concatenate([edge[0], npd % n]).reshape(_NW, kc, _CH)
    dst_ch = jnp.concatenate([edge[1], n + npd % nscr]).reshape(_NW, kc, _CH)

    batchc = batch.reshape(n, 1)
    rpt = npad // _NS
    z1 = jnp.zeros((npad,), jnp.float32)
    zr = jnp.zeros((_NS, rpt, h), jnp.float32)

    b1r = b1.reshape(1, h)
    b2r = b2.reshape(1, h)
    gr = gamma.reshape(1, h)
    ber = beta.reshape(1, h)
    blr = bl.reshape(1, c)

    bm = 2000

    deg = _deg_call(npad, kc)(dst_ch, z1)
    degc = (deg[0, :n] + deg[1, :n] + 1.0).reshape(n, 1)
    hs1 = _mm_scale_call(n, npad, d, h, bm)(x, W1, degc)
    acc1 = _spmm_call(npad, kc, h)(src_ch, dst_ch, hs1, zr)
    hs2 = _post_mm_call(n, npad, h, bm)(acc1, hs1, degc, b1r, gr, ber, W2)
    acc2 = _spmm_call(npad, kc, h)(src_ch, dst_ch, hs2, zr)
    out = _post_pool_call(n, npad, h, c, bm)(
        acc2, hs2, degc, b2r, gr, ber, batchc, Wl, blr)
    return out
